# baseline (speedup 1.0000x reference)
import functools

import jax
import jax.numpy as jnp
import numpy as np
from jax import lax
from jax.experimental import pallas as pl
from jax.experimental.pallas import tpu as pltpu
from jax.experimental.pallas import tpu_sc as plsc

B = 2
LQ = 5440
C = 256
NH = 8
NL = 4
NP = 4
HD = C // NH
SHAPES = [(64, 64), (32, 32), (16, 16), (8, 8)]
LEVEL_START = [0, 4096, 5120, 5376]
LIN = sum(h * w for h, w in SHAPES)

KPQ = 64
QB = 544
NQB = LQ // QB
QW = LQ // 2
QBLK = 16
NBLK = QW // QBLK
ROWS = QBLK * KPQ

_lane = np.arange(NH * NL * NP)
_l_of = (_lane % (NL * NP)) // NP
_h_of = _lane // (NL * NP)
_WL = np.array([SHAPES[l][1] for l in _l_of], np.float32)
_HL = np.array([SHAPES[l][0] for l in _l_of], np.float32)
_WLI = _WL.astype(np.int32)
_BASE = np.array([LEVEL_START[l] for l in _l_of], np.int32)
_S8X = np.zeros((2 * NL, 128), np.float32)
_S8X[2 * _l_of, _lane] = 1.0
_S8Y = np.zeros((2 * NL, 128), np.float32)
_S8Y[2 * _l_of + 1, _lane] = 1.0
_SOFX = np.zeros((2 * NH * NL * NP, 128), np.float32)
_SOFX[2 * _lane, _lane] = 1.0
_SOFY = np.zeros((2 * NH * NL * NP, 128), np.float32)
_SOFY[2 * _lane + 1, _lane] = 1.0
_G16 = np.kron(np.eye(NH, dtype=np.float32), np.ones((16, 16), np.float32))


def _matmul_bias_kernel(x_ref, w_ref, b_ref, o_ref):
    o_ref[...] = (
        jnp.dot(x_ref[...], w_ref[...].T, preferred_element_type=jnp.float32)
        + b_ref[...][None, :]
    )


def _matmul_bias(x, w, b, blk=640):
    M, K = x.shape
    N = w.shape[0]
    return pl.pallas_call(
        _matmul_bias_kernel,
        grid=(M // blk,),
        in_specs=[
            pl.BlockSpec((blk, K), lambda i: (i, 0)),
            pl.BlockSpec((N, K), lambda i: (0, 0)),
            pl.BlockSpec((N,), lambda i: (0,)),
        ],
        out_specs=pl.BlockSpec((blk, N), lambda i: (i, 0)),
        out_shape=jax.ShapeDtypeStruct((M, N), jnp.float32),
    )(x, w, b)


def _prep_kernel(q_ref, r_ref, woff_ref, boff_ref, wattn_ref, battn_ref,
                 wl_ref, hl_ref, wli_ref, base_ref, s8x_ref, s8y_ref,
                 sofx_ref, sofy_ref, g16_ref, idx_ref, wt_ref):
    b = pl.program_id(0)
    wl = wl_ref[...][None, :]
    hl = hl_ref[...][None, :]
    wli = wli_ref[...][None, :]
    q = q_ref[0]
    off = (jnp.dot(q, woff_ref[...].T, preferred_element_type=jnp.float32)
           + boff_ref[...][None, :])
    logits = (jnp.dot(q, wattn_ref[...].T, preferred_element_type=jnp.float32)
              + battn_ref[...][None, :])
    m = jnp.max(logits, axis=-1, keepdims=True)
    e = jnp.exp(logits - m)
    hi = jax.lax.Precision.HIGHEST
    denom = jnp.dot(e, g16_ref[...], preferred_element_type=jnp.float32,
                    precision=hi)
    attn = e / denom
    r8 = r_ref[0].reshape(QB, 2 * NL)
    rx = jnp.dot(r8, s8x_ref[...], preferred_element_type=jnp.float32,
                 precision=hi)
    ry = jnp.dot(r8, s8y_ref[...], preferred_element_type=jnp.float32,
                 precision=hi)
    offx = jnp.dot(off, sofx_ref[...], preferred_element_type=jnp.float32,
                   precision=hi)
    offy = jnp.dot(off, sofy_ref[...], preferred_element_type=jnp.float32,
                   precision=hi)
    xs = rx * wl + offx - 0.5
    ys = ry * hl + offy - 0.5
    x0 = jnp.floor(xs)
    y0 = jnp.floor(ys)
    x1 = x0 + 1.0
    y1 = y0 + 1.0
    del b
    base = base_ref[...][None, :]
    corners = (
        (x0, y0, (x1 - xs) * (y1 - ys)),
        (x0, y1, (x1 - xs) * (ys - y0)),
        (x1, y0, (xs - x0) * (y1 - ys)),
        (x1, y1, (xs - x0) * (ys - y0)),
    )
    for ci, (cx, cy, wgt) in enumerate(corners):
        valid = ((cx >= 0.0) & (cx <= wl - 1.0)
                 & (cy >= 0.0) & (cy <= hl - 1.0))
        w = attn * wgt * valid.astype(jnp.float32)
        ix = jnp.clip(cx, 0.0, wl - 1.0).astype(jnp.int32)
        iy = jnp.clip(cy, 0.0, hl - 1.0).astype(jnp.int32)
        idx = (base + iy * wli + ix) * (HD // 2)
        idx_ref[0, :, ci, :] = idx
        wt_ref[0, :, ci, :] = w


def _sampling_prep(query, reference_points, W_off, b_off, W_attn, b_attn):
    return pl.pallas_call(
        _prep_kernel,
        grid=(B, NQB),
        in_specs=[
            pl.BlockSpec((1, QB, C), lambda b, i: (b, i, 0)),
            pl.BlockSpec((1, QB, NL, 2), lambda b, i: (b, i, 0, 0)),
            pl.BlockSpec((2 * NH * NL * NP, C), lambda b, i: (0, 0)),
            pl.BlockSpec((2 * NH * NL * NP,), lambda b, i: (0,)),
            pl.BlockSpec((NH * NL * NP, C), lambda b, i: (0, 0)),
            pl.BlockSpec((NH * NL * NP,), lambda b, i: (0,)),
            pl.BlockSpec((128,), lambda b, i: (0,)),
            pl.BlockSpec((128,), lambda b, i: (0,)),
            pl.BlockSpec((128,), lambda b, i: (0,)),
            pl.BlockSpec((128,), lambda b, i: (0,)),
            pl.BlockSpec((2 * NL, 128), lambda b, i: (0, 0)),
            pl.BlockSpec((2 * NL, 128), lambda b, i: (0, 0)),
            pl.BlockSpec((2 * NH * NL * NP, 128), lambda b, i: (0, 0)),
            pl.BlockSpec((2 * NH * NL * NP, 128), lambda b, i: (0, 0)),
            pl.BlockSpec((128, 128), lambda b, i: (0, 0)),
        ],
        out_specs=[
            pl.BlockSpec((1, QB, 4, 128), lambda b, i: (b, i, 0, 0)),
            pl.BlockSpec((1, QB, 4, 128), lambda b, i: (b, i, 0, 0)),
        ],
        out_shape=[
            jax.ShapeDtypeStruct((B, LQ, 4, 128), jnp.int32),
            jax.ShapeDtypeStruct((B, LQ, 4, 128), jnp.float32),
        ],
    )(query, reference_points, W_off, b_off, W_attn, b_attn,
      jnp.asarray(_WL), jnp.asarray(_HL), jnp.asarray(_WLI),
      jnp.asarray(_BASE), jnp.asarray(_S8X), jnp.asarray(_S8Y),
      jnp.asarray(_SOFX), jnp.asarray(_SOFY), jnp.asarray(_G16))


NWORD = HD // 2
TABW = LIN * NWORD
OBLK = QBLK * HD


SLAB = QBLK * 4 * 128


def _sc_body(tab, idxs, wts, out, tab_v, idx0, idx1, wt0, wt1, o0, o1,
             sem_t, sem_i, sem_w, sem_o):
    wid = lax.axis_index("s") * 2 + lax.axis_index("c")
    b = wid // 16
    rest = wid % 16
    h = rest // 2
    half = rest % 2
    bh = wid // 2
    qb = half * QW
    iota16 = lax.iota(jnp.int32, 16)
    iota_h = iota16 * 512 + h * 16
    iota_o = iota16 * HD

    pltpu.async_copy(tab.at[bh], tab_v, sem_t)

    def slab_slice(j):
        return pl.ds(pl.multiple_of((qb + j * QBLK) * 512, 8), SLAB)

    def start_ic(j, idxv, wtv):
        pltpu.async_copy(idxs.at[b, slab_slice(j)], idxv, sem_i)
        pltpu.async_copy(wts.at[b, slab_slice(j)], wtv, sem_w)

    def wait_ic(j, idxv, wtv):
        pltpu.make_async_copy(idxs.at[b, slab_slice(j)], idxv, sem_i).wait()
        pltpu.make_async_copy(wts.at[b, slab_slice(j)], wtv, sem_w).wait()

    def out_slice(j):
        return out.at[bh, pl.ds(pl.multiple_of((qb + j * QBLK) * HD, 8), OBLK)]

    def compute(idxv, wtv, ov):
        def kbody(k, acc):
            rows = iota_h + k
            wk = plsc.load_gather(wtv, [rows])
            basek = plsc.load_gather(idxv, [rows])
            accl = list(acc)
            for r in range(8):
                rowsr = basek + r if r else basek
                for s in range(2):
                    c2 = s * 8 + r
                    ref = tab_v if s == 0 else tab_v.at[pl.ds(8, TABW - 8)]
                    w32 = plsc.load_gather(ref, [rowsr])
                    pair = plsc.bitcast(w32, jnp.bfloat16)
                    lo, hi = plsc.unpack(pair,
                                         format=plsc.PackFormat.INTERLEAVED)
                    accl[2 * c2] = accl[2 * c2] + wk * lo
                    accl[2 * c2 + 1] = accl[2 * c2 + 1] + wk * hi
            return tuple(accl)

        def cbody(c, acc):
            return lax.fori_loop(c * 128, c * 128 + 16, kbody, acc)

        acc = lax.fori_loop(
            0, 4, cbody,
            tuple(jnp.zeros((16,), jnp.float32) for _ in range(HD)))
        for c in range(HD):
            plsc.store_scatter(ov, [iota_o + c], acc[c])

    def step(j, idxv, wtv, ov):
        wait_ic(j, idxv, wtv)

        @pl.when(j >= 2)
        def _():
            pltpu.make_async_copy(ov, out_slice(j - 2), sem_o).wait()

        compute(idxv, wtv, ov)
        pltpu.async_copy(ov, out_slice(j), sem_o)

        @pl.when(j + 2 < NBLK)
        def _():
            start_ic(j + 2, idxv, wtv)

    start_ic(0, idx0, wt0)
    start_ic(1, idx1, wt1)
    pltpu.make_async_copy(tab.at[bh], tab_v, sem_t).wait()

    def pair(i, carry):
        j0 = i * 2
        step(j0, idx0, wt0, o0)
        step(j0 + 1, idx1, wt1, o1)
        return carry

    lax.fori_loop(0, NBLK // 2, pair, 0)
    pltpu.make_async_copy(o0, out_slice(NBLK - 2), sem_o).wait()
    pltpu.make_async_copy(o1, out_slice(NBLK - 1), sem_o).wait()


def _sc_sample(tab, idxs, wts):
    fn = pl.kernel(
        _sc_body,
        out_type=jax.ShapeDtypeStruct((B * NH, LQ * HD), jnp.float32),
        mesh=plsc.VectorSubcoreMesh(core_axis_name="c", subcore_axis_name="s",
                                    num_cores=2, num_subcores=16),
        compiler_params=pltpu.CompilerParams(needs_layout_passes=False,
                                             use_tc_tiling_on_sc=False),
        scratch_types=[
            pltpu.VMEM((TABW,), jnp.int32),
            pltpu.VMEM((SLAB,), jnp.int32),
            pltpu.VMEM((SLAB,), jnp.int32),
            pltpu.VMEM((SLAB,), jnp.float32),
            pltpu.VMEM((SLAB,), jnp.float32),
            pltpu.VMEM((OBLK,), jnp.float32),
            pltpu.VMEM((OBLK,), jnp.float32),
            pltpu.SemaphoreType.DMA,
            pltpu.SemaphoreType.DMA,
            pltpu.SemaphoreType.DMA,
            pltpu.SemaphoreType.DMA,
        ],
    )
    return fn(tab, idxs, wts)


def kernel(query, reference_points, input_flatten, input_spatial_shapes,
           input_level_start_index, W_off, b_off, W_attn, b_attn, W_val, b_val,
           W_out, b_out):
    value = _matmul_bias(input_flatten.reshape(B * LIN, C), W_val, b_val)
    vb = (value.reshape(B, LIN, NH, HD).transpose(0, 2, 1, 3)
          .astype(jnp.bfloat16))
    tab = jax.lax.bitcast_convert_type(
        vb.reshape(B, NH, LIN, NWORD, 2), jnp.int32).reshape(B * NH, TABW)
    idx4, wt4 = _sampling_prep(query, reference_points, W_off, b_off,
                               W_attn, b_attn)
    idxs = idx4.reshape(B, LQ * 4 * 128)
    wts = wt4.reshape(B, LQ * 4 * 128)
    z = _sc_sample(tab, idxs, wts)
    z = (z.reshape(B, NH, LQ, HD).transpose(0, 2, 1, 3).reshape(B * LQ, C))
    return _matmul_bias(z, W_out, b_out).reshape(B, LQ, C)


# speedup vs baseline: 1.2326x; 1.2326x over previous
import functools

import jax
import jax.numpy as jnp
import numpy as np
from jax import lax
from jax.experimental import pallas as pl
from jax.experimental.pallas import tpu as pltpu
from jax.experimental.pallas import tpu_sc as plsc

B = 2
LQ = 5440
C = 256
NH = 8
NL = 4
NP = 4
HD = C // NH
SHAPES = [(64, 64), (32, 32), (16, 16), (8, 8)]
LEVEL_START = [0, 4096, 5120, 5376]
LIN = sum(h * w for h, w in SHAPES)

KPQ = 64
QB = 544
NQB = LQ // QB
QW = LQ // 2
QBLK = 16
NBLK = QW // QBLK
ROWS = QBLK * KPQ

_lane = np.arange(NH * NL * NP)
_l_of = (_lane % (NL * NP)) // NP
_h_of = _lane // (NL * NP)
_WL = np.array([SHAPES[l][1] for l in _l_of], np.float32)
_HL = np.array([SHAPES[l][0] for l in _l_of], np.float32)
_WLI = _WL.astype(np.int32)
_BASE = np.array([LEVEL_START[l] for l in _l_of], np.int32)
_S8X = np.zeros((2 * NL, 128), np.float32)
_S8X[2 * _l_of, _lane] = 1.0
_S8Y = np.zeros((2 * NL, 128), np.float32)
_S8Y[2 * _l_of + 1, _lane] = 1.0
_SOFX = np.zeros((2 * NH * NL * NP, 128), np.float32)
_SOFX[2 * _lane, _lane] = 1.0
_SOFY = np.zeros((2 * NH * NL * NP, 128), np.float32)
_SOFY[2 * _lane + 1, _lane] = 1.0
_G16 = np.kron(np.eye(NH, dtype=np.float32), np.ones((16, 16), np.float32))


def _matmul_bias_kernel(x_ref, w_ref, b_ref, o_ref):
    o_ref[...] = (
        jnp.dot(x_ref[...], w_ref[...].T, preferred_element_type=jnp.float32)
        + b_ref[...][None, :]
    )


def _matmul_bias(x, w, b, blk=640):
    M, K = x.shape
    N = w.shape[0]
    return pl.pallas_call(
        _matmul_bias_kernel,
        grid=(M // blk,),
        in_specs=[
            pl.BlockSpec((blk, K), lambda i: (i, 0)),
            pl.BlockSpec((N, K), lambda i: (0, 0)),
            pl.BlockSpec((N,), lambda i: (0,)),
        ],
        out_specs=pl.BlockSpec((blk, N), lambda i: (i, 0)),
        out_shape=jax.ShapeDtypeStruct((M, N), jnp.float32),
    )(x, w, b)


def _prep_kernel(q_ref, r_ref, woff_ref, boff_ref, wattn_ref, battn_ref,
                 wl_ref, hl_ref, wli_ref, base_ref, s8x_ref, s8y_ref,
                 sofx_ref, sofy_ref, g16_ref, idx_ref, wt_ref):
    b = pl.program_id(0)
    wl = wl_ref[...][None, :]
    hl = hl_ref[...][None, :]
    wli = wli_ref[...][None, :]
    q = q_ref[0]
    off = (jnp.dot(q, woff_ref[...].T, preferred_element_type=jnp.float32)
           + boff_ref[...][None, :])
    logits = (jnp.dot(q, wattn_ref[...].T, preferred_element_type=jnp.float32)
              + battn_ref[...][None, :])
    m = jnp.max(logits, axis=-1, keepdims=True)
    e = jnp.exp(logits - m)
    hi = jax.lax.Precision.HIGHEST
    denom = jnp.dot(e, g16_ref[...], preferred_element_type=jnp.float32,
                    precision=hi)
    attn = e / denom
    r8 = r_ref[0].reshape(QB, 2 * NL)
    rx = jnp.dot(r8, s8x_ref[...], preferred_element_type=jnp.float32,
                 precision=hi)
    ry = jnp.dot(r8, s8y_ref[...], preferred_element_type=jnp.float32,
                 precision=hi)
    offx = jnp.dot(off, sofx_ref[...], preferred_element_type=jnp.float32,
                   precision=hi)
    offy = jnp.dot(off, sofy_ref[...], preferred_element_type=jnp.float32,
                   precision=hi)
    xs = rx * wl + offx - 0.5
    ys = ry * hl + offy - 0.5
    x0 = jnp.floor(xs)
    y0 = jnp.floor(ys)
    x1 = x0 + 1.0
    y1 = y0 + 1.0
    del b
    base = base_ref[...][None, :]
    corners = (
        (x0, y0, (x1 - xs) * (y1 - ys)),
        (x0, y1, (x1 - xs) * (ys - y0)),
        (x1, y0, (xs - x0) * (y1 - ys)),
        (x1, y1, (xs - x0) * (ys - y0)),
    )
    for ci, (cx, cy, wgt) in enumerate(corners):
        valid = ((cx >= 0.0) & (cx <= wl - 1.0)
                 & (cy >= 0.0) & (cy <= hl - 1.0))
        w = attn * wgt * valid.astype(jnp.float32)
        ix = jnp.clip(cx, 0.0, wl - 1.0).astype(jnp.int32)
        iy = jnp.clip(cy, 0.0, hl - 1.0).astype(jnp.int32)
        idx = (base + iy * wli + ix) * (HD // 2)
        idx_ref[0, :, ci, :] = idx
        wt_ref[0, :, ci, :] = w


def _sampling_prep(query, reference_points, W_off, b_off, W_attn, b_attn):
    return pl.pallas_call(
        _prep_kernel,
        grid=(B, NQB),
        in_specs=[
            pl.BlockSpec((1, QB, C), lambda b, i: (b, i, 0)),
            pl.BlockSpec((1, QB, NL, 2), lambda b, i: (b, i, 0, 0)),
            pl.BlockSpec((2 * NH * NL * NP, C), lambda b, i: (0, 0)),
            pl.BlockSpec((2 * NH * NL * NP,), lambda b, i: (0,)),
            pl.BlockSpec((NH * NL * NP, C), lambda b, i: (0, 0)),
            pl.BlockSpec((NH * NL * NP,), lambda b, i: (0,)),
            pl.BlockSpec((128,), lambda b, i: (0,)),
            pl.BlockSpec((128,), lambda b, i: (0,)),
            pl.BlockSpec((128,), lambda b, i: (0,)),
            pl.BlockSpec((128,), lambda b, i: (0,)),
            pl.BlockSpec((2 * NL, 128), lambda b, i: (0, 0)),
            pl.BlockSpec((2 * NL, 128), lambda b, i: (0, 0)),
            pl.BlockSpec((2 * NH * NL * NP, 128), lambda b, i: (0, 0)),
            pl.BlockSpec((2 * NH * NL * NP, 128), lambda b, i: (0, 0)),
            pl.BlockSpec((128, 128), lambda b, i: (0, 0)),
        ],
        out_specs=[
            pl.BlockSpec((1, QB, 4, 128), lambda b, i: (b, i, 0, 0)),
            pl.BlockSpec((1, QB, 4, 128), lambda b, i: (b, i, 0, 0)),
        ],
        out_shape=[
            jax.ShapeDtypeStruct((B, LQ, 4, 128), jnp.int32),
            jax.ShapeDtypeStruct((B, LQ, 4, 128), jnp.float32),
        ],
    )(query, reference_points, W_off, b_off, W_attn, b_attn,
      jnp.asarray(_WL), jnp.asarray(_HL), jnp.asarray(_WLI),
      jnp.asarray(_BASE), jnp.asarray(_S8X), jnp.asarray(_S8Y),
      jnp.asarray(_SOFX), jnp.asarray(_SOFY), jnp.asarray(_G16))


NWORD = HD // 2
TABW = LIN * NWORD
OBLK = QBLK * HD


SLAB = QBLK * 4 * 128


def _sc_body(tab, idxs, wts, out, tab_v, idx0, idx1, wt0, wt1, o0, o1,
             sem_t, sem_i, sem_w, sem_o):
    wid = lax.axis_index("s") * 2 + lax.axis_index("c")
    b = wid // 16
    rest = wid % 16
    h = rest // 2
    half = rest % 2
    bh = wid // 2
    qb = half * QW
    iota16 = lax.iota(jnp.int32, 16)
    iota_h = iota16 * 512 + h * 16
    iota_o = iota16 * HD

    pltpu.async_copy(tab.at[bh], tab_v, sem_t)

    def slab_slice(j):
        return pl.ds(pl.multiple_of((qb + j * QBLK) * 512, 8), SLAB)

    def start_ic(j, idxv, wtv):
        pltpu.async_copy(idxs.at[b, slab_slice(j)], idxv, sem_i)
        pltpu.async_copy(wts.at[b, slab_slice(j)], wtv, sem_w)

    def wait_ic(j, idxv, wtv):
        pltpu.make_async_copy(idxs.at[b, slab_slice(j)], idxv, sem_i).wait()
        pltpu.make_async_copy(wts.at[b, slab_slice(j)], wtv, sem_w).wait()

    def out_slice(j):
        return out.at[bh, pl.ds(pl.multiple_of((qb + j * QBLK) * HD, 8), OBLK)]

    def compute(idxv, wtv, ov):
        def kbody(k, acc):
            rows = iota16 + k
            wk = plsc.load_gather(wtv, [rows])
            basek = plsc.load_gather(idxv, [rows])
            accl = list(acc)
            for r in range(8):
                rowsr = basek + r if r else basek
                for s in range(2):
                    c2 = s * 8 + r
                    ref = tab_v if s == 0 else tab_v.at[pl.ds(8, TABW - 8)]
                    w32 = plsc.load_gather(ref, [rowsr])
                    pair = plsc.bitcast(w32, jnp.bfloat16)
                    lo, hi = plsc.unpack(pair,
                                         format=plsc.PackFormat.INTERLEAVED)
                    accl[2 * c2] = accl[2 * c2] + wk * lo
                    accl[2 * c2 + 1] = accl[2 * c2 + 1] + wk * hi
            return tuple(accl)

        def cbody(c, acc):
            return lax.fori_loop(c * 128, c * 128 + 16, kbody, acc)

        acc = lax.fori_loop(
            0, 4, cbody,
            tuple(jnp.zeros((16,), jnp.float32) for _ in range(HD)))
        for c in range(HD):
            plsc.store_scatter(ov, [iota_o + c], acc[c])

    def step(j, idxv, wtv, ov):
        wait_ic(j, idxv, wtv)

        @pl.when(j >= 2)
        def _():
            pltpu.make_async_copy(ov, out_slice(j - 2), sem_o).wait()

        compute(idxv, wtv, ov)
        pltpu.async_copy(ov, out_slice(j), sem_o)

        @pl.when(j + 2 < NBLK)
        def _():
            start_ic(j + 2, idxv, wtv)

    start_ic(0, idx0, wt0)
    start_ic(1, idx1, wt1)
    pltpu.make_async_copy(tab.at[bh], tab_v, sem_t).wait()

    def pair(i, carry):
        j0 = i * 2
        step(j0, idx0, wt0, o0)
        step(j0 + 1, idx1, wt1, o1)
        return carry

    lax.fori_loop(0, NBLK // 2, pair, 0)
    pltpu.make_async_copy(o0, out_slice(NBLK - 2), sem_o).wait()
    pltpu.make_async_copy(o1, out_slice(NBLK - 1), sem_o).wait()


def _sc_sample(tab, idxs, wts):
    fn = pl.kernel(
        _sc_body,
        out_type=jax.ShapeDtypeStruct((B * NH, LQ * HD), jnp.float32),
        mesh=plsc.VectorSubcoreMesh(core_axis_name="c", subcore_axis_name="s",
                                    num_cores=2, num_subcores=16),
        compiler_params=pltpu.CompilerParams(needs_layout_passes=False,
                                             use_tc_tiling_on_sc=False),
        scratch_types=[
            pltpu.VMEM((TABW,), jnp.int32),
            pltpu.VMEM((SLAB,), jnp.int32),
            pltpu.VMEM((SLAB,), jnp.int32),
            pltpu.VMEM((SLAB,), jnp.float32),
            pltpu.VMEM((SLAB,), jnp.float32),
            pltpu.VMEM((OBLK,), jnp.float32),
            pltpu.VMEM((OBLK,), jnp.float32),
            pltpu.SemaphoreType.DMA,
            pltpu.SemaphoreType.DMA,
            pltpu.SemaphoreType.DMA,
            pltpu.SemaphoreType.DMA,
        ],
    )
    return fn(tab, idxs, wts)


def kernel(query, reference_points, input_flatten, input_spatial_shapes,
           input_level_start_index, W_off, b_off, W_attn, b_attn, W_val, b_val,
           W_out, b_out):
    value = _matmul_bias(input_flatten.reshape(B * LIN, C), W_val, b_val)
    vb = (value.reshape(B, LIN, NH, HD).transpose(0, 2, 1, 3)
          .astype(jnp.bfloat16))
    tab = jax.lax.bitcast_convert_type(
        vb.reshape(B, NH, LIN, NWORD, 2), jnp.int32).reshape(B * NH, TABW)
    idx4, wt4 = _sampling_prep(query, reference_points, W_off, b_off,
                               W_attn, b_attn)
    idxs = idx4.reshape(B, LQ * 4 * 128)
    wts = wt4.reshape(B, LQ * 4 * 128)
    z = _sc_sample(tab, idxs, wts)
    z = (z.reshape(B, NH, LQ, HD).transpose(0, 2, 1, 3).reshape(B * LQ, C))
    return _matmul_bias(z, W_out, b_out).reshape(B, LQ, C)
